# Initial kernel scaffold; baseline (speedup 1.0000x reference)
import jax
import jax.numpy as jnp
from jax.experimental import pallas as pl


def kernel(x, edge_index, W1, b1, W2, b2):
    raise NotImplementedError("write your pallas kernel here")


# speedup vs baseline: 14.6610x; 14.6610x over previous
import functools

import jax
import jax.numpy as jnp
from jax import lax
from jax.experimental import pallas as pl
from jax.experimental.pallas import tpu as pltpu
from jax.experimental.pallas import tpu_sc as plsc

NC = 2
NS = 16
NW = NC * NS
K = 80
DEG_W = 16

@functools.cache
def _mesh():
    return plsc.VectorSubcoreMesh(core_axis_name="c", subcore_axis_name="s")


_SC_PARAMS = pltpu.CompilerParams(use_tc_tiling_on_sc=False)


def _zero_spmem(zbuf, acc_sh, sid, rows_per_sub, zr, width):
    @pl.loop(0, zr)
    def _(i):
        for j in range(width // 16):
            zbuf[i, pl.ds(j * 16, 16)] = jnp.zeros((16,), jnp.float32)

    for r in range(rows_per_sub // zr):
        pltpu.sync_copy(zbuf, acc_sh.at[pl.ds(sid * rows_per_sub + r * zr, zr)])


def _pad_rows(n):
    unit = NS * 8 * 16
    return -(-n // unit) * unit


def _make_deg_call(n, e):
    epw = e // NW
    chunks = epw // K
    npad = _pad_rows(n)
    rps = npad // NS
    zr = min(rps, 128)
    assert rps % zr == 0

    @functools.partial(
        pl.kernel,
        mesh=_mesh(),
        out_type=jax.ShapeDtypeStruct((NC, npad, DEG_W), jnp.float32),
        scratch_types=[
            pltpu.VMEM((K,), jnp.int32),
            pltpu.VMEM((K, DEG_W), jnp.float32),
            pltpu.VMEM((zr, DEG_W), jnp.float32),
            pltpu.VMEM_SHARED((npad, DEG_W), jnp.float32),
            pltpu.SemaphoreType.DMA,
        ],
        compiler_params=_SC_PARAMS,
    )
    def deg_call(dst_hbm, out_hbm, idx_v, ones_v, zbuf, acc_sh, sem):
        cid = lax.axis_index("c")
        sid = lax.axis_index("s")
        wid = sid * NC + cid

        @pl.loop(0, K)
        def _(i):
            ones_v[i, pl.ds(0, 16)] = jnp.full((16,), 1.0, jnp.float32)

        _zero_spmem(zbuf, acc_sh, sid, rps, zr, DEG_W)
        plsc.subcore_barrier()

        @pl.loop(0, chunks)
        def _(c):
            base = wid * epw + c * K
            pltpu.sync_copy(dst_hbm.at[pl.ds(base, K)], idx_v)
            pltpu.sync_copy(ones_v, acc_sh.at[idx_v], add=True)

        plsc.subcore_barrier()
        pltpu.sync_copy(
            acc_sh.at[pl.ds(sid * rps, rps)],
            out_hbm.at[cid, pl.ds(sid * rps, rps)],
        )

    return deg_call


def _make_edge_call(n, e, hid):
    epw = e // NW
    chunks = epw // K
    npad = _pad_rows(n)
    rps = npad // NS
    zr = min(rps, 128)
    assert rps % zr == 0

    @functools.partial(
        pl.kernel,
        mesh=_mesh(),
        out_type=jax.ShapeDtypeStruct((NC, npad, hid), jnp.float32),
        scratch_types=[
            pltpu.VMEM((K,), jnp.int32),
            pltpu.VMEM((K,), jnp.int32),
            pltpu.VMEM((K, hid), jnp.float32),
            pltpu.VMEM((zr, hid), jnp.float32),
            pltpu.VMEM_SHARED((npad, hid), jnp.float32),
            pltpu.SemaphoreType.DMA,
        ],
        compiler_params=_SC_PARAMS,
    )
    def edge_call(hs_hbm, src_hbm, dst_hbm, out_hbm,
                  sidx_v, didx_v, rows_v, zbuf, acc_sh, sem):
        cid = lax.axis_index("c")
        sid = lax.axis_index("s")
        wid = sid * NC + cid

        _zero_spmem(zbuf, acc_sh, sid, rps, zr, hid)
        plsc.subcore_barrier()

        @pl.loop(0, chunks)
        def _(c):
            base = wid * epw + c * K
            pltpu.sync_copy(src_hbm.at[pl.ds(base, K)], sidx_v)
            pltpu.sync_copy(dst_hbm.at[pl.ds(base, K)], didx_v)
            pltpu.async_copy(hs_hbm.at[sidx_v], rows_v, sem).wait()
            pltpu.sync_copy(rows_v, acc_sh.at[didx_v], add=True)

        plsc.subcore_barrier()
        pltpu.sync_copy(
            acc_sh.at[pl.ds(sid * rps, rps)],
            out_hbm.at[cid, pl.ds(sid * rps, rps)],
        )

    return edge_call


def _deg_dis(dacc_blk):
    deg = dacc_blk[0][:, 0:1] + dacc_blk[1][:, 0:1] + 1.0
    dis = lax.rsqrt(deg)
    return dis, dis * dis


def _k1_body(dacc_ref, x_ref, w1_ref, h_ref, hs_ref):
    dis, _ = _deg_dis(dacc_ref)
    h = jnp.dot(x_ref[...], w1_ref[...], preferred_element_type=jnp.float32)
    h_ref[...] = h
    hs_ref[...] = h * dis


def _k2_body(dacc_ref, acc_ref, h1_ref, w2_ref, b1_ref, h2_ref, h2s_ref):
    dis, dinv = _deg_dis(dacc_ref)
    z = dis * (acc_ref[0] + acc_ref[1]) + dinv * h1_ref[...] + b1_ref[...]
    a = jnp.where(z >= 0, z, 0.01 * z)
    h2 = jnp.dot(a, w2_ref[...], preferred_element_type=jnp.float32)
    h2_ref[...] = h2
    h2s_ref[...] = h2 * dis


def _k3_body(dacc_ref, acc_ref, h2_ref, b2_ref, out_ref):
    dis, dinv = _deg_dis(dacc_ref)
    z = dis * (acc_ref[0] + acc_ref[1]) + dinv * h2_ref[...] + b2_ref[...]
    out_ref[...] = jnp.where(z >= 0, z, 0.01 * z)


def kernel(x, edge_index, W1, b1, W2, b2):
    n, in_ch = x.shape
    e = edge_index.shape[1]
    hid = W1.shape[1]
    rb = 1000
    grid = (n // rb,)

    src = edge_index[0].astype(jnp.int32)
    dst = edge_index[1].astype(jnp.int32)

    deg_call = _make_deg_call(n, e)
    edge_call = _make_edge_call(n, e, hid)

    dacc = deg_call(dst)

    dacc_spec = pl.BlockSpec((NC, rb, DEG_W), lambda i: (0, i, 0))
    acc_spec = pl.BlockSpec((NC, rb, hid), lambda i: (0, i, 0))
    row_spec = pl.BlockSpec((rb, hid), lambda i: (i, 0))
    bias_spec = pl.BlockSpec((1, hid), lambda i: (0, 0))

    h1, h1s = pl.pallas_call(
        _k1_body,
        grid=grid,
        in_specs=[
            dacc_spec,
            pl.BlockSpec((rb, in_ch), lambda i: (i, 0)),
            pl.BlockSpec((in_ch, hid), lambda i: (0, 0)),
        ],
        out_specs=[row_spec, row_spec],
        out_shape=[
            jax.ShapeDtypeStruct((n, hid), jnp.float32),
            jax.ShapeDtypeStruct((n, hid), jnp.float32),
        ],
    )(dacc, x, W1)

    acc1 = edge_call(h1s, src, dst)

    h2, h2s = pl.pallas_call(
        _k2_body,
        grid=grid,
        in_specs=[
            dacc_spec,
            acc_spec,
            row_spec,
            pl.BlockSpec((hid, hid), lambda i: (0, 0)),
            bias_spec,
        ],
        out_specs=[row_spec, row_spec],
        out_shape=[
            jax.ShapeDtypeStruct((n, hid), jnp.float32),
            jax.ShapeDtypeStruct((n, hid), jnp.float32),
        ],
    )(dacc, acc1, h1, W2, b1.reshape(1, hid))

    acc2 = edge_call(h2s, src, dst)

    out = pl.pallas_call(
        _k3_body,
        grid=grid,
        in_specs=[dacc_spec, acc_spec, row_spec, bias_spec],
        out_specs=row_spec,
        out_shape=jax.ShapeDtypeStruct((n, hid), jnp.float32),
    )(dacc, acc2, h2, b2.reshape(1, hid))

    return out


# speedup vs baseline: 43.6763x; 2.9791x over previous
import functools

import jax
import jax.numpy as jnp
from jax import lax
from jax.experimental import pallas as pl
from jax.experimental.pallas import tpu as pltpu
from jax.experimental.pallas import tpu_sc as plsc

NC = 2
NS = 16
NW = NC * NS
K = 80
NBUF = 5
DEG_W = 16

@functools.cache
def _mesh():
    return plsc.VectorSubcoreMesh(core_axis_name="c", subcore_axis_name="s")


_SC_PARAMS = pltpu.CompilerParams(use_tc_tiling_on_sc=False)


def _zero_spmem(zbuf, acc_sh, sid, rows_per_sub, zr, width):
    @pl.loop(0, zr)
    def _(i):
        for j in range(width // 16):
            zbuf[i, pl.ds(j * 16, 16)] = jnp.zeros((16,), jnp.float32)

    for r in range(rows_per_sub // zr):
        pltpu.sync_copy(zbuf, acc_sh.at[pl.ds(sid * rows_per_sub + r * zr, zr)])


def _pad_rows(n):
    unit = NS * 8 * 16
    return -(-n // unit) * unit


def _make_deg_call(n, e):
    epw = e // NW
    chunks = epw // K
    npad = _pad_rows(n)
    rps = npad // NS
    zr = min(rps, 128)
    assert rps % zr == 0

    @functools.partial(
        pl.kernel,
        mesh=_mesh(),
        out_type=jax.ShapeDtypeStruct((NC, npad, DEG_W), jnp.float32),
        scratch_types=[
            pltpu.VMEM((chunks, K), jnp.int32),
            pltpu.VMEM((K, DEG_W), jnp.float32),
            pltpu.VMEM((zr, DEG_W), jnp.float32),
            pltpu.VMEM_SHARED((npad, DEG_W), jnp.float32),
            pltpu.SemaphoreType.DMA,
        ],
        compiler_params=_SC_PARAMS,
    )
    def deg_call(dst_hbm, out_hbm, idx_v, ones_v, zbuf, acc_sh, sem):
        cid = lax.axis_index("c")
        sid = lax.axis_index("s")
        wid = sid * NC + cid

        @pl.loop(0, K)
        def _(i):
            ones_v[i, pl.ds(0, 16)] = jnp.full((16,), 1.0, jnp.float32)

        pltpu.async_copy(dst_hbm.at[wid], idx_v, sem)
        _zero_spmem(zbuf, acc_sh, sid, rps, zr, DEG_W)
        pltpu.make_async_copy(dst_hbm.at[wid], idx_v, sem).wait()
        plsc.subcore_barrier()

        @pl.loop(0, chunks)
        def _(c):
            pltpu.sync_copy(ones_v, acc_sh.at[idx_v.at[c]], add=True)

        plsc.subcore_barrier()
        pltpu.sync_copy(
            acc_sh.at[pl.ds(sid * rps, rps)],
            out_hbm.at[cid, pl.ds(sid * rps, rps)],
        )

    return deg_call


def _make_edge_call(n, e, hid):
    epw = e // NW
    chunks = epw // K
    npad = _pad_rows(n)
    rps = npad // NS
    zr = min(rps, 128)
    assert rps % zr == 0

    @functools.partial(
        pl.kernel,
        mesh=_mesh(),
        out_type=jax.ShapeDtypeStruct((NC, npad, hid), jnp.float32),
        scratch_types=[
            pltpu.VMEM((chunks, K), jnp.int32),
            pltpu.VMEM((chunks, K), jnp.int32),
            [pltpu.VMEM((K, hid), jnp.float32) for _ in range(NBUF)],
            pltpu.VMEM((zr, hid), jnp.float32),
            pltpu.VMEM_SHARED((npad, hid), jnp.float32),
            [pltpu.SemaphoreType.DMA for _ in range(NBUF)],
            pltpu.SemaphoreType.DMA,
        ],
        compiler_params=_SC_PARAMS,
    )
    def edge_call(hs_hbm, src_hbm, dst_hbm, out_hbm,
                  sidx_v, didx_v, rows, zbuf, acc_sh, gsems, isem):
        cid = lax.axis_index("c")
        sid = lax.axis_index("s")
        wid = sid * NC + cid

        pltpu.async_copy(src_hbm.at[wid], sidx_v, isem)
        pltpu.async_copy(dst_hbm.at[wid], didx_v, isem)
        _zero_spmem(zbuf, acc_sh, sid, rps, zr, hid)
        pltpu.make_async_copy(src_hbm.at[wid], sidx_v, isem).wait()
        pltpu.make_async_copy(dst_hbm.at[wid], didx_v, isem).wait()

        for b in range(NBUF):
            pltpu.async_copy(hs_hbm.at[sidx_v.at[b]], rows[b], gsems[b])

        plsc.subcore_barrier()

        @pl.loop(0, chunks, step=NBUF)
        def _(g):
            for b in range(NBUF):
                c = g + b
                pltpu.make_async_copy(
                    hs_hbm.at[sidx_v.at[c]], rows[b], gsems[b]).wait()
                pltpu.sync_copy(rows[b], acc_sh.at[didx_v.at[c]], add=True)
                nxt = c + NBUF

                @pl.when(nxt < chunks)
                def _():
                    pltpu.async_copy(
                        hs_hbm.at[sidx_v.at[nxt]], rows[b], gsems[b])

        plsc.subcore_barrier()
        pltpu.sync_copy(
            acc_sh.at[pl.ds(sid * rps, rps)],
            out_hbm.at[cid, pl.ds(sid * rps, rps)],
        )

    return edge_call


def _deg_dis(dacc_blk):
    deg = dacc_blk[0][:, 0:1] + dacc_blk[1][:, 0:1] + 1.0
    dis = lax.rsqrt(deg)
    return dis, dis * dis


def _k1_body(dacc_ref, x_ref, w1_ref, h_ref, hs_ref):
    dis, _ = _deg_dis(dacc_ref)
    h = jnp.dot(x_ref[...], w1_ref[...], preferred_element_type=jnp.float32)
    h_ref[...] = h
    hs_ref[...] = h * dis


def _k2_body(dacc_ref, acc_ref, h1_ref, w2_ref, b1_ref, h2_ref, h2s_ref):
    dis, dinv = _deg_dis(dacc_ref)
    z = dis * (acc_ref[0] + acc_ref[1]) + dinv * h1_ref[...] + b1_ref[...]
    a = jnp.where(z >= 0, z, 0.01 * z)
    h2 = jnp.dot(a, w2_ref[...], preferred_element_type=jnp.float32)
    h2_ref[...] = h2
    h2s_ref[...] = h2 * dis


def _k3_body(dacc_ref, acc_ref, h2_ref, b2_ref, out_ref):
    dis, dinv = _deg_dis(dacc_ref)
    z = dis * (acc_ref[0] + acc_ref[1]) + dinv * h2_ref[...] + b2_ref[...]
    out_ref[...] = jnp.where(z >= 0, z, 0.01 * z)


def kernel(x, edge_index, W1, b1, W2, b2):
    n, in_ch = x.shape
    e = edge_index.shape[1]
    hid = W1.shape[1]
    rb = 1000
    grid = (n // rb,)

    epw = e // NW
    chunks = epw // K
    assert epw % K == 0 and chunks % NBUF == 0
    src = edge_index[0].astype(jnp.int32).reshape(NW, chunks, K)
    dst = edge_index[1].astype(jnp.int32).reshape(NW, chunks, K)

    deg_call = _make_deg_call(n, e)
    edge_call = _make_edge_call(n, e, hid)

    dacc = deg_call(dst)

    dacc_spec = pl.BlockSpec((NC, rb, DEG_W), lambda i: (0, i, 0))
    acc_spec = pl.BlockSpec((NC, rb, hid), lambda i: (0, i, 0))
    row_spec = pl.BlockSpec((rb, hid), lambda i: (i, 0))
    bias_spec = pl.BlockSpec((1, hid), lambda i: (0, 0))

    h1, h1s = pl.pallas_call(
        _k1_body,
        grid=grid,
        in_specs=[
            dacc_spec,
            pl.BlockSpec((rb, in_ch), lambda i: (i, 0)),
            pl.BlockSpec((in_ch, hid), lambda i: (0, 0)),
        ],
        out_specs=[row_spec, row_spec],
        out_shape=[
            jax.ShapeDtypeStruct((n, hid), jnp.float32),
            jax.ShapeDtypeStruct((n, hid), jnp.float32),
        ],
    )(dacc, x, W1)

    acc1 = edge_call(h1s, src, dst)

    h2, h2s = pl.pallas_call(
        _k2_body,
        grid=grid,
        in_specs=[
            dacc_spec,
            acc_spec,
            row_spec,
            pl.BlockSpec((hid, hid), lambda i: (0, 0)),
            bias_spec,
        ],
        out_specs=[row_spec, row_spec],
        out_shape=[
            jax.ShapeDtypeStruct((n, hid), jnp.float32),
            jax.ShapeDtypeStruct((n, hid), jnp.float32),
        ],
    )(dacc, acc1, h1, W2, b1.reshape(1, hid))

    acc2 = edge_call(h2s, src, dst)

    out = pl.pallas_call(
        _k3_body,
        grid=grid,
        in_specs=[dacc_spec, acc_spec, row_spec, bias_spec],
        out_specs=row_spec,
        out_shape=jax.ShapeDtypeStruct((n, hid), jnp.float32),
    )(dacc, acc2, h2, b2.reshape(1, hid))

    return out
